# Initial kernel scaffold; baseline (speedup 1.0000x reference)
import jax
import jax.numpy as jnp
from jax.experimental import pallas as pl


def kernel(x, edge_index, W1, b1, gamma, beta, W2, b2):
    raise NotImplementedError("write your pallas kernel here")


# speedup vs baseline: 18.8843x; 18.8843x over previous
import functools

import jax
import jax.numpy as jnp
from jax import lax
from jax.experimental import pallas as pl
from jax.experimental.pallas import tpu as pltpu
from jax.experimental.pallas import tpu_sc as plsc

N = 10000
E = 320000
D_IN = 128
D_HID = 128
K = 16

NC = 2
NS = 16
NW = NC * NS
EPW = E // NW
CH = 80
NCHUNK = EPW // CH
NPAD = 10240
RPT = NPAD // NS
RCH = 128
NRCH = RPT // RCH

_MESH = plsc.VectorSubcoreMesh(core_axis_name="c", subcore_axis_name="s")


def _zero_rows(buf, rows, d):
    z16 = jnp.zeros((16,), jnp.float32)
    for g in range(d // 16):
        def body(r, _, g=g):
            buf[r, pl.ds(16 * g, 16)] = z16
            return 0
        lax.fori_loop(0, rows, body, 0)


def _make_agg(d, nhalf, use_table):
    scratch = [
        pltpu.VMEM((NCHUNK, CH), jnp.int32),
        pltpu.VMEM((NCHUNK, CH), jnp.int32),
        pltpu.VMEM((CH, d), jnp.float32),
        pltpu.VMEM((RCH, d), jnp.float32),
        pltpu.VMEM_SHARED((NPAD, d), jnp.float32),
    ]

    def body(*refs):
        if use_table:
            tables = refs[:nhalf]
            src_hbm, dst_hbm, out_hbm = refs[nhalf:nhalf + 3]
        else:
            src_hbm, dst_hbm, out_hbm = refs[:3]
        src_v, dst_v, rows_v, chunk_v, acc_sh = refs[-5:]

        c = lax.axis_index("c")
        s = lax.axis_index("s")
        wid = s * NC + c

        _zero_rows(chunk_v, RCH, d)
        if not use_table:
            one16 = jnp.ones((16,), jnp.float32)
            for g in range(d // 16):
                def obody(r, _, g=g):
                    rows_v[r, pl.ds(16 * g, 16)] = one16
                    return 0
                lax.fori_loop(0, CH, obody, 0)

        if use_table:
            pltpu.sync_copy(src_hbm.at[wid], src_v)
        pltpu.sync_copy(dst_hbm.at[wid], dst_v)

        base = s * RPT
        for h in range(nhalf):
            for k in range(NRCH):
                pltpu.sync_copy(chunk_v, acc_sh.at[pl.ds(base + k * RCH, RCH)])
            plsc.subcore_barrier()

            def edge_body(j, _, h=h):
                if use_table:
                    pltpu.sync_copy(tables[h].at[src_v.at[j]], rows_v)
                pltpu.sync_copy(rows_v, acc_sh.at[dst_v.at[j]], add=True)
                return 0
            lax.fori_loop(0, NCHUNK, edge_body, 0)
            plsc.subcore_barrier()

            for k in range(NRCH):
                pltpu.sync_copy(acc_sh.at[pl.ds(base + k * RCH, RCH)], chunk_v)
                pltpu.sync_copy(chunk_v,
                                out_hbm.at[c, h, pl.ds(base + k * RCH, RCH)])
            if h + 1 < nhalf:
                _zero_rows(chunk_v, RCH, d)

    return functools.partial(
        pl.kernel, body, mesh=_MESH,
        out_type=jax.ShapeDtypeStruct((NC, nhalf, NPAD, d), jnp.float32),
        scratch_types=scratch,
        compiler_params=pltpu.CompilerParams(use_tc_tiling_on_sc=False),
    )()


def _dinv_col(degp):
    deg = degp[0, 0] + degp[1, 0] + 1.0
    return lax.rsqrt(deg)[:N, 0:1]


def _tc1_body(x_ref, w1_ref, degp_ref, th1_ref):
    dinv = _dinv_col(degp_ref[...])
    th1_ref[...] = dinv * jnp.dot(x_ref[...], w1_ref[...],
                                  preferred_element_type=jnp.float32)


def _tc2_body(acc_ref, th1_ref, degp_ref, b1_ref, gamma_ref, beta_ref,
              w2_ref, th2_ref):
    dinv = _dinv_col(degp_ref[...])
    accsum = jnp.concatenate(
        [acc_ref[0, 0, :N, :] + acc_ref[1, 0, :N, :],
         acc_ref[0, 1, :N, :] + acc_ref[1, 1, :N, :]], axis=-1)
    h1 = dinv * (accsum + th1_ref[...])
    h1 = h1 + b1_ref[...]
    mean = jnp.mean(h1, axis=0)
    var = jnp.mean((h1 - mean) ** 2, axis=0)
    z = (h1 - mean) * lax.rsqrt(var + 1e-5) * gamma_ref[...] + beta_ref[...]
    z = jnp.maximum(z, 0.0)
    th2_ref[...] = dinv * jnp.dot(z, w2_ref[...],
                                  preferred_element_type=jnp.float32)


def _tc3_body(acc_ref, th2_ref, degp_ref, b2_ref, out_ref):
    dinv = _dinv_col(degp_ref[...])
    h2 = dinv * (acc_ref[0, 0, :N, :] + acc_ref[1, 0, :N, :] + th2_ref[...])
    h2 = h2 + b2_ref[...]
    e = jnp.exp(h2 - jnp.max(h2, axis=-1, keepdims=True))
    out_ref[...] = e / jnp.sum(e, axis=-1, keepdims=True)


def kernel(x, edge_index, W1, b1, gamma, beta, W2, b2):
    src_r = edge_index[0].reshape(NW, NCHUNK, CH)
    dst_r = edge_index[1].reshape(NW, NCHUNK, CH)

    degp = _make_agg(16, 1, use_table=False)(src_r, dst_r)

    th1 = pl.pallas_call(
        _tc1_body,
        out_shape=jax.ShapeDtypeStruct((N, D_HID), jnp.float32),
    )(x, W1, degp)

    th1_lo = th1[:, :D_HID // 2]
    th1_hi = th1[:, D_HID // 2:]
    acc1 = _make_agg(D_HID // 2, 2, use_table=True)(th1_lo, th1_hi,
                                                    src_r, dst_r)

    th2 = pl.pallas_call(
        _tc2_body,
        out_shape=jax.ShapeDtypeStruct((N, K), jnp.float32),
    )(acc1, th1, degp, b1, gamma, beta, W2)

    acc2 = _make_agg(K, 1, use_table=True)(th2, src_r, dst_r)

    out = pl.pallas_call(
        _tc3_body,
        out_shape=jax.ShapeDtypeStruct((N, K), jnp.float32),
    )(acc2, th2, degp, b2)

    return out


# speedup vs baseline: 31.6771x; 1.6774x over previous
import functools

import jax
import jax.numpy as jnp
from jax import lax
from jax.experimental import pallas as pl
from jax.experimental.pallas import tpu as pltpu
from jax.experimental.pallas import tpu_sc as plsc

N = 10000
E = 320000
D_IN = 128
D_HID = 128
K = 16

NC = 2
NS = 16
NW = NC * NS
EPW = E // NW
CH = 125
NCHUNK = EPW // CH
GK = 2
NG = NCHUNK // GK
NPAD = 10240
RPT = NPAD // NS
RCH = 128
NRCH = RPT // RCH

_MESH = plsc.VectorSubcoreMesh(core_axis_name="c", subcore_axis_name="s")


def _zero_rows(buf, rows, d):
    z16 = jnp.zeros((16,), jnp.float32)
    for g in range(d // 16):
        def body(r, _, g=g):
            buf[r, pl.ds(16 * g, 16)] = z16
            return 0
        lax.fori_loop(0, rows, body, 0)


def _make_agg(d, nhalf, use_table):
    scratch = [
        pltpu.VMEM((NCHUNK, CH), jnp.int32),
        pltpu.VMEM((NCHUNK, CH), jnp.int32),
        pltpu.VMEM((2, GK, CH, d), jnp.float32),
        pltpu.VMEM((RCH, d), jnp.float32),
        pltpu.VMEM_SHARED((NPAD, d), jnp.float32),
        pltpu.SemaphoreType.DMA((2,)),
        pltpu.SemaphoreType.DMA((2,)),
    ]

    def body(*refs):
        if use_table:
            tables = refs[:nhalf]
            src_hbm, dst_hbm, out_hbm = refs[nhalf:nhalf + 3]
        else:
            src_hbm, dst_hbm, out_hbm = refs[:3]
        src_v, dst_v, rows_v, chunk_v, acc_sh, gsem, ssem = refs[-7:]

        c = lax.axis_index("c")
        s = lax.axis_index("s")
        wid = s * NC + c

        _zero_rows(chunk_v, RCH, d)
        if not use_table:
            one16 = jnp.ones((16,), jnp.float32)
            for g in range(d // 16):
                def obody(r, _, g=g):
                    rows_v[0, 0, r, pl.ds(16 * g, 16)] = one16
                    return 0
                lax.fori_loop(0, CH, obody, 0)

        if use_table:
            pltpu.sync_copy(src_hbm.at[wid], src_v)
        pltpu.sync_copy(dst_hbm.at[wid], dst_v)

        def gat(h, jj, g, i):
            return pltpu.make_async_copy(
                tables[h].at[src_v.at[jj]], rows_v.at[g, i], gsem.at[g])

        def sca(jj, g, i):
            src = rows_v.at[0, 0] if not use_table else rows_v.at[g, i]
            return pltpu.make_async_copy(
                src, acc_sh.at[dst_v.at[jj]], ssem.at[g])

        base = s * RPT
        for h in range(nhalf):
            for k in range(NRCH):
                pltpu.sync_copy(chunk_v, acc_sh.at[pl.ds(base + k * RCH, RCH)])
            plsc.subcore_barrier()

            if use_table:
                for i in range(GK):
                    gat(h, i, 0, i).start()

            def pair_body(j2, _, h=h):
                for g in range(2):
                    jg = j2 + g
                    cb = jg * GK
                    if use_table:
                        for i in range(GK):
                            gat(h, cb + i, g, i).wait()
                    for i in range(GK):
                        sca(cb + i, g, i).start(add=True)
                    go = 1 - g
                    @pl.when(jg > 0)
                    def _():
                        for i in range(GK):
                            sca(cb - GK + i, go, i).wait()
                    if use_table:
                        @pl.when(jg + 1 < NG)
                        def _():
                            for i in range(GK):
                                gat(h, cb + GK + i, go, i).start()
                return 0
            lax.fori_loop(0, NG // 2, lambda t, u: pair_body(2 * t, u), 0)
            for i in range(GK):
                sca((NG - 1) * GK + i, 1, i).wait()
            plsc.subcore_barrier()

            for k in range(NRCH):
                pltpu.sync_copy(acc_sh.at[pl.ds(base + k * RCH, RCH)], chunk_v)
                pltpu.sync_copy(chunk_v,
                                out_hbm.at[c, h, pl.ds(base + k * RCH, RCH)])
            if h + 1 < nhalf:
                _zero_rows(chunk_v, RCH, d)

    return functools.partial(
        pl.kernel, body, mesh=_MESH,
        out_type=jax.ShapeDtypeStruct((NC, nhalf, NPAD, d), jnp.float32),
        scratch_types=scratch,
        compiler_params=pltpu.CompilerParams(use_tc_tiling_on_sc=False),
    )()


def _dinv_col(degp):
    deg = degp[0, 0] + degp[1, 0] + 1.0
    return lax.rsqrt(deg)[:N, 0:1]


def _tc1_body(x_ref, w1_ref, degp_ref, th1_ref):
    dinv = _dinv_col(degp_ref[...])
    th1_ref[...] = dinv * jnp.dot(x_ref[...], w1_ref[...],
                                  preferred_element_type=jnp.float32)


def _tc2_body(acc_ref, th1_ref, degp_ref, b1_ref, gamma_ref, beta_ref,
              w2_ref, th2_ref):
    dinv = _dinv_col(degp_ref[...])
    accsum = jnp.concatenate(
        [acc_ref[0, 0, :N, :] + acc_ref[1, 0, :N, :],
         acc_ref[0, 1, :N, :] + acc_ref[1, 1, :N, :]], axis=-1)
    h1 = dinv * (accsum + th1_ref[...])
    h1 = h1 + b1_ref[...]
    mean = jnp.mean(h1, axis=0)
    var = jnp.mean((h1 - mean) ** 2, axis=0)
    z = (h1 - mean) * lax.rsqrt(var + 1e-5) * gamma_ref[...] + beta_ref[...]
    z = jnp.maximum(z, 0.0)
    th2_ref[...] = dinv * jnp.dot(z, w2_ref[...],
                                  preferred_element_type=jnp.float32)


def _tc3_body(acc_ref, th2_ref, degp_ref, b2_ref, out_ref):
    dinv = _dinv_col(degp_ref[...])
    h2 = dinv * (acc_ref[0, 0, :N, :] + acc_ref[1, 0, :N, :] + th2_ref[...])
    h2 = h2 + b2_ref[...]
    e = jnp.exp(h2 - jnp.max(h2, axis=-1, keepdims=True))
    out_ref[...] = e / jnp.sum(e, axis=-1, keepdims=True)


def kernel(x, edge_index, W1, b1, gamma, beta, W2, b2):
    src_r = edge_index[0].reshape(NW, NCHUNK, CH)
    dst_r = edge_index[1].reshape(NW, NCHUNK, CH)

    degp = _make_agg(16, 1, use_table=False)(src_r, dst_r)

    th1 = pl.pallas_call(
        _tc1_body,
        out_shape=jax.ShapeDtypeStruct((N, D_HID), jnp.float32),
    )(x, W1, degp)

    th1_lo = th1[:, :D_HID // 2]
    th1_hi = th1[:, D_HID // 2:]
    acc1 = _make_agg(D_HID // 2, 2, use_table=True)(th1_lo, th1_hi,
                                                    src_r, dst_r)

    th2 = pl.pallas_call(
        _tc2_body,
        out_shape=jax.ShapeDtypeStruct((N, K), jnp.float32),
    )(acc1, th1, degp, b1, gamma, beta, W2)

    acc2 = _make_agg(K, 1, use_table=True)(th2, src_r, dst_r)

    out = pl.pallas_call(
        _tc3_body,
        out_shape=jax.ShapeDtypeStruct((N, K), jnp.float32),
    )(acc2, th2, degp, b2)

    return out


# speedup vs baseline: 35.7774x; 1.1294x over previous
import functools

import jax
import jax.numpy as jnp
from jax import lax
from jax.experimental import pallas as pl
from jax.experimental.pallas import tpu as pltpu
from jax.experimental.pallas import tpu_sc as plsc

N = 10000
E = 320000
D_IN = 128
D_HID = 128
K = 16

NC = 2
NS = 16
NW = NC * NS
EPW = E // NW
CH = 100
NCHUNK = EPW // CH
RING = 5
LEAD = 3
NPAD = 10240
RPT = NPAD // NS
RCH = 128
NRCH = RPT // RCH

_MESH = plsc.VectorSubcoreMesh(core_axis_name="c", subcore_axis_name="s")


def _zero_rows(buf, rows, d):
    z16 = jnp.zeros((16,), jnp.float32)
    for g in range(d // 16):
        def body(r, _, g=g):
            buf[r, pl.ds(16 * g, 16)] = z16
            return 0
        lax.fori_loop(0, rows, body, 0)


def _make_agg(d, nhalf, use_table):
    scratch = [
        pltpu.VMEM((NCHUNK, CH), jnp.int32),
        pltpu.VMEM((NCHUNK, CH), jnp.int32),
        pltpu.VMEM((RING, CH, d), jnp.float32),
        pltpu.VMEM((RCH, d), jnp.float32),
        pltpu.VMEM_SHARED((NPAD, d), jnp.float32),
        pltpu.SemaphoreType.DMA((RING,)),
        pltpu.SemaphoreType.DMA((RING,)),
    ]

    def body(*refs):
        if use_table:
            tables = refs[:nhalf]
            src_hbm, dst_hbm, out_hbm = refs[nhalf:nhalf + 3]
        else:
            src_hbm, dst_hbm, out_hbm = refs[:3]
        src_v, dst_v, rows_v, chunk_v, acc_sh, gsem, ssem = refs[-7:]

        c = lax.axis_index("c")
        s = lax.axis_index("s")
        wid = s * NC + c

        _zero_rows(chunk_v, RCH, d)
        if not use_table:
            one16 = jnp.ones((16,), jnp.float32)
            for g in range(d // 16):
                def obody(r, _, g=g):
                    rows_v[0, r, pl.ds(16 * g, 16)] = one16
                    return 0
                lax.fori_loop(0, CH, obody, 0)

        if use_table:
            pltpu.sync_copy(src_hbm.at[wid], src_v)
        pltpu.sync_copy(dst_hbm.at[wid], dst_v)

        def gat(h, jj, q):
            return pltpu.make_async_copy(
                tables[h].at[src_v.at[jj]], rows_v.at[q], gsem.at[q])

        def sca(jj, q):
            src = rows_v.at[0] if not use_table else rows_v.at[q]
            return pltpu.make_async_copy(
                src, acc_sh.at[dst_v.at[jj]], ssem.at[q])

        base = s * RPT
        for h in range(nhalf):
            for k in range(NRCH):
                pltpu.sync_copy(chunk_v, acc_sh.at[pl.ds(base + k * RCH, RCH)])
            plsc.subcore_barrier()

            if use_table:
                for q in range(LEAD):
                    gat(h, q, q).start()

            def rev_body(t, _, h=h):
                for q in range(RING):
                    jj = RING * t + q
                    if use_table:
                        gat(h, jj, q).wait()
                    sca(jj, q).start(add=True)
                    @pl.when(jj >= 2)
                    def _():
                        sca(jj - 2, (q - 2) % RING).wait()
                    if use_table:
                        @pl.when(jj + LEAD < NCHUNK)
                        def _():
                            gat(h, jj + LEAD, (q + LEAD) % RING).start()
                return 0
            lax.fori_loop(0, NCHUNK // RING, rev_body, 0)
            sca(NCHUNK - 2, (NCHUNK - 2) % RING).wait()
            sca(NCHUNK - 1, (NCHUNK - 1) % RING).wait()
            plsc.subcore_barrier()

            for k in range(NRCH):
                pltpu.sync_copy(acc_sh.at[pl.ds(base + k * RCH, RCH)], chunk_v)
                pltpu.sync_copy(chunk_v,
                                out_hbm.at[c, h, pl.ds(base + k * RCH, RCH)])
            if h + 1 < nhalf:
                _zero_rows(chunk_v, RCH, d)

    return functools.partial(
        pl.kernel, body, mesh=_MESH,
        out_type=jax.ShapeDtypeStruct((NC, nhalf, NPAD, d), jnp.float32),
        scratch_types=scratch,
        compiler_params=pltpu.CompilerParams(use_tc_tiling_on_sc=False),
    )()


def _dinv_col(degp):
    deg = degp[0, 0] + degp[1, 0] + 1.0
    return lax.rsqrt(deg)[:N, 0:1]


def _tc1_body(x_ref, w1_ref, degp_ref, lo_ref, hi_ref):
    dinv = _dinv_col(degp_ref[...])
    th1 = dinv * jnp.dot(x_ref[...], w1_ref[...],
                         preferred_element_type=jnp.float32)
    lo_ref[...] = th1[:, :D_HID // 2]
    hi_ref[...] = th1[:, D_HID // 2:]


def _tc2_body(acc_ref, lo_ref, hi_ref, degp_ref, b1_ref, gamma_ref,
              beta_ref, w2_ref, th2_ref):
    dinv = _dinv_col(degp_ref[...])
    accsum = jnp.concatenate(
        [acc_ref[0, 0, :N, :] + acc_ref[1, 0, :N, :] + lo_ref[...],
         acc_ref[0, 1, :N, :] + acc_ref[1, 1, :N, :] + hi_ref[...]],
        axis=-1)
    h1 = dinv * accsum
    h1 = h1 + b1_ref[...]
    mean = jnp.mean(h1, axis=0)
    var = jnp.mean((h1 - mean) ** 2, axis=0)
    z = (h1 - mean) * lax.rsqrt(var + 1e-5) * gamma_ref[...] + beta_ref[...]
    z = jnp.maximum(z, 0.0)
    th2_ref[...] = dinv * jnp.dot(z, w2_ref[...],
                                  preferred_element_type=jnp.float32)


def _tc3_body(acc_ref, th2_ref, degp_ref, b2_ref, out_ref):
    dinv = _dinv_col(degp_ref[...])
    h2 = dinv * (acc_ref[0, 0, :N, :] + acc_ref[1, 0, :N, :] + th2_ref[...])
    h2 = h2 + b2_ref[...]
    e = jnp.exp(h2 - jnp.max(h2, axis=-1, keepdims=True))
    out_ref[...] = e / jnp.sum(e, axis=-1, keepdims=True)


def kernel(x, edge_index, W1, b1, gamma, beta, W2, b2):
    src_r = edge_index[0].reshape(NW, NCHUNK, CH)
    dst_r = edge_index[1].reshape(NW, NCHUNK, CH)

    degp = _make_agg(16, 1, use_table=False)(src_r, dst_r)

    th1_lo, th1_hi = pl.pallas_call(
        _tc1_body,
        out_shape=[jax.ShapeDtypeStruct((N, D_HID // 2), jnp.float32),
                   jax.ShapeDtypeStruct((N, D_HID // 2), jnp.float32)],
    )(x, W1, degp)

    acc1 = _make_agg(D_HID // 2, 2, use_table=True)(th1_lo, th1_hi,
                                                    src_r, dst_r)

    th2 = pl.pallas_call(
        _tc2_body,
        out_shape=jax.ShapeDtypeStruct((N, K), jnp.float32),
    )(acc1, th1_lo, th1_hi, degp, b1, gamma, beta, W2)

    acc2 = _make_agg(K, 1, use_table=True)(th2, src_r, dst_r)

    out = pl.pallas_call(
        _tc3_body,
        out_shape=jax.ShapeDtypeStruct((N, K), jnp.float32),
    )(acc2, th2, degp, b2)

    return out


# speedup vs baseline: 38.6791x; 1.0811x over previous
import functools

import jax
import jax.numpy as jnp
from jax import lax
from jax.experimental import pallas as pl
from jax.experimental.pallas import tpu as pltpu
from jax.experimental.pallas import tpu_sc as plsc

N = 10000
E = 320000
D_IN = 128
D_HID = 128
K = 16

NC = 2
NS = 16
NW = NC * NS
EPW = E // NW
CH = 125
NCHUNK = EPW // CH
NPAD = 10240
RPT = NPAD // NS
RCH = 128
NRCH = RPT // RCH

_MESH = plsc.VectorSubcoreMesh(core_axis_name="c", subcore_axis_name="s")


def _zero_rows(buf, rows, d):
    z16 = jnp.zeros((16,), jnp.float32)
    for g in range(d // 16):
        def body(r, _, g=g):
            buf[r, pl.ds(16 * g, 16)] = z16
            return 0
        lax.fori_loop(0, rows, body, 0)


def _make_agg(d, nhalf, use_table, ring, lead):
    drain = ring - lead
    scratch = [
        pltpu.VMEM((NCHUNK, CH), jnp.int32),
        pltpu.VMEM((NCHUNK, CH), jnp.int32),
        pltpu.VMEM((ring, CH, d), jnp.float32),
        pltpu.VMEM((RCH, d), jnp.float32),
        pltpu.VMEM_SHARED((NPAD, d), jnp.float32),
        pltpu.SemaphoreType.DMA((ring,)),
        pltpu.SemaphoreType.DMA((ring,)),
    ]

    def body(*refs):
        if use_table:
            tables = refs[:nhalf]
            er_hbm, out_hbm = refs[nhalf:nhalf + 2]
        else:
            er_hbm, out_hbm = refs[:2]
        src_v, dst_v, rows_v, chunk_v, acc_sh, gsem, ssem = refs[-7:]

        c = lax.axis_index("c")
        s = lax.axis_index("s")
        wid = s * NC + c

        _zero_rows(chunk_v, RCH, d)
        if not use_table:
            one16 = jnp.ones((16,), jnp.float32)
            for g in range(d // 16):
                def obody(r, _, g=g):
                    rows_v[0, r, pl.ds(16 * g, 16)] = one16
                    return 0
                lax.fori_loop(0, CH, obody, 0)

        if use_table:
            pltpu.sync_copy(er_hbm.at[0, wid], src_v)
        pltpu.sync_copy(er_hbm.at[1, wid], dst_v)

        def gat(h, jj, q):
            return pltpu.make_async_copy(
                tables[h].at[src_v.at[jj]], rows_v.at[q], gsem.at[q])

        def sca(jj, q):
            src = rows_v.at[0] if not use_table else rows_v.at[q]
            return pltpu.make_async_copy(
                src, acc_sh.at[dst_v.at[jj]], ssem.at[q])

        base = s * RPT
        for h in range(nhalf):
            for k in range(NRCH):
                pltpu.sync_copy(chunk_v, acc_sh.at[pl.ds(base + k * RCH, RCH)])
            plsc.subcore_barrier()

            if use_table:
                for q in range(lead):
                    gat(h, q, q).start()

            def rev_body(t, _, h=h):
                for q in range(ring):
                    jj = ring * t + q
                    if use_table:
                        gat(h, jj, q).wait()
                    sca(jj, q).start(add=True)
                    @pl.when(jj >= drain)
                    def _():
                        sca(jj - drain, (q - drain) % ring).wait()
                    if use_table:
                        @pl.when(jj + lead < NCHUNK)
                        def _():
                            gat(h, jj + lead, (q + lead) % ring).start()
                return 0
            lax.fori_loop(0, NCHUNK // ring, rev_body, 0)
            for k in range(drain):
                jj = NCHUNK - drain + k
                sca(jj, jj % ring).wait()
            plsc.subcore_barrier()

            for k in range(NRCH):
                pltpu.sync_copy(acc_sh.at[pl.ds(base + k * RCH, RCH)], chunk_v)
                pltpu.sync_copy(chunk_v,
                                out_hbm.at[c, h, pl.ds(base + k * RCH, RCH)])
            if h + 1 < nhalf:
                _zero_rows(chunk_v, RCH, d)

    return functools.partial(
        pl.kernel, body, mesh=_MESH,
        out_type=jax.ShapeDtypeStruct((NC, nhalf, NPAD, d), jnp.float32),
        scratch_types=scratch,
        compiler_params=pltpu.CompilerParams(use_tc_tiling_on_sc=False),
    )()


def _dinv_col(degp):
    deg = degp[0, 0] + degp[1, 0] + 1.0
    return lax.rsqrt(deg)[:N, 0:1]


def _tc1_body(x_ref, w1_ref, degp_ref, lo_ref, hi_ref):
    dinv = _dinv_col(degp_ref[...])
    th1 = dinv * jnp.dot(x_ref[...], w1_ref[...],
                         preferred_element_type=jnp.float32)
    lo_ref[...] = th1[:, :D_HID // 2]
    hi_ref[...] = th1[:, D_HID // 2:]


def _tc2_body(acc_ref, lo_ref, hi_ref, degp_ref, b1_ref, gamma_ref,
              beta_ref, w2_ref, th2_ref):
    dinv = _dinv_col(degp_ref[...])
    accsum = jnp.concatenate(
        [acc_ref[0, 0, :N, :] + acc_ref[1, 0, :N, :] + lo_ref[...],
         acc_ref[0, 1, :N, :] + acc_ref[1, 1, :N, :] + hi_ref[...]],
        axis=-1)
    h1 = dinv * accsum
    h1 = h1 + b1_ref[...]
    mean = jnp.mean(h1, axis=0)
    var = jnp.mean((h1 - mean) ** 2, axis=0)
    z = (h1 - mean) * lax.rsqrt(var + 1e-5) * gamma_ref[...] + beta_ref[...]
    z = jnp.maximum(z, 0.0)
    th2_ref[...] = dinv * jnp.dot(z, w2_ref[...],
                                  preferred_element_type=jnp.float32)


def _tc3_body(acc_ref, th2_ref, degp_ref, b2_ref, out_ref):
    dinv = _dinv_col(degp_ref[...])
    h2 = dinv * (acc_ref[0, 0, :N, :] + acc_ref[1, 0, :N, :] + th2_ref[...])
    h2 = h2 + b2_ref[...]
    e = jnp.exp(h2 - jnp.max(h2, axis=-1, keepdims=True))
    out_ref[...] = e / jnp.sum(e, axis=-1, keepdims=True)


def kernel(x, edge_index, W1, b1, gamma, beta, W2, b2):
    er = edge_index.reshape(2, NW, NCHUNK, CH)

    degp = _make_agg(16, 1, use_table=False, ring=8, lead=4)(er)

    th1_lo, th1_hi = pl.pallas_call(
        _tc1_body,
        out_shape=[jax.ShapeDtypeStruct((N, D_HID // 2), jnp.float32),
                   jax.ShapeDtypeStruct((N, D_HID // 2), jnp.float32)],
    )(x, W1, degp)

    acc1 = _make_agg(D_HID // 2, 2, use_table=True, ring=5, lead=3)(
        th1_lo, th1_hi, er)

    th2 = pl.pallas_call(
        _tc2_body,
        out_shape=jax.ShapeDtypeStruct((N, K), jnp.float32),
    )(acc1, th1_lo, th1_hi, degp, b1, gamma, beta, W2)

    acc2 = _make_agg(K, 1, use_table=True, ring=8, lead=4)(th2, er)

    out = pl.pallas_call(
        _tc3_body,
        out_shape=jax.ShapeDtypeStruct((N, K), jnp.float32),
    )(acc2, th2, degp, b2)

    return out


# speedup vs baseline: 39.5188x; 1.0217x over previous
import functools

import jax
import jax.numpy as jnp
from jax import lax
from jax.experimental import pallas as pl
from jax.experimental.pallas import tpu as pltpu
from jax.experimental.pallas import tpu_sc as plsc

N = 10000
E = 320000
D_IN = 128
D_HID = 128
K = 16

NC = 2
NS = 16
NW = NC * NS
EPW = E // NW
CH = 125
NCHUNK = EPW // CH
NPAD = 10240
RPT = NPAD // NS
RCH = 128
NRCH = RPT // RCH

_MESH = plsc.VectorSubcoreMesh(core_axis_name="c", subcore_axis_name="s")


def _zero_rows(buf, rows, d):
    z16 = jnp.zeros((16,), jnp.float32)
    for g in range(d // 16):
        def body(r, _, g=g):
            buf[r, pl.ds(16 * g, 16)] = z16
            return 0
        lax.fori_loop(0, rows, body, 0)


def _make_agg(d, nhalf, use_table, ring, lead):
    drain = ring - lead
    scratch = [
        pltpu.VMEM((NCHUNK, CH), jnp.int32),
        pltpu.VMEM((NCHUNK, CH), jnp.int32),
        pltpu.VMEM((ring, CH, d), jnp.float32),
        pltpu.VMEM((RCH, d), jnp.float32),
        pltpu.VMEM_SHARED((NPAD, d), jnp.float32),
        pltpu.SemaphoreType.DMA((ring,)),
        pltpu.SemaphoreType.DMA((ring,)),
    ]

    def body(*refs):
        if use_table:
            tables = refs[:nhalf]
            er_hbm, out_hbm = refs[nhalf:nhalf + 2]
        else:
            er_hbm, out_hbm = refs[:2]
        src_v, dst_v, rows_v, chunk_v, acc_sh, gsem, ssem = refs[-7:]

        c = lax.axis_index("c")
        s = lax.axis_index("s")
        wid = s * NC + c

        _zero_rows(chunk_v, RCH, d)
        if not use_table:
            one16 = jnp.ones((16,), jnp.float32)
            for g in range(d // 16):
                def obody(r, _, g=g):
                    rows_v[0, r, pl.ds(16 * g, 16)] = one16
                    return 0
                lax.fori_loop(0, CH, obody, 0)

        if use_table:
            pltpu.sync_copy(er_hbm.at[0, wid], src_v)
        pltpu.sync_copy(er_hbm.at[1, wid], dst_v)

        def gat(h, jj, q):
            return pltpu.make_async_copy(
                tables[h].at[src_v.at[jj]], rows_v.at[q], gsem.at[q])

        def sca(jj, q):
            src = rows_v.at[0] if not use_table else rows_v.at[q]
            return pltpu.make_async_copy(
                src, acc_sh.at[dst_v.at[jj]], ssem.at[q])

        base = s * RPT
        for h in range(nhalf):
            for k in range(NRCH):
                pltpu.sync_copy(chunk_v, acc_sh.at[pl.ds(base + k * RCH, RCH)])
            plsc.subcore_barrier()

            if use_table:
                for q in range(lead):
                    gat(h, q, q).start()

            def rev_body(t, _, h=h):
                for q in range(ring):
                    jj = ring * t + q
                    if use_table:
                        gat(h, jj, q).wait()
                    sca(jj, q).start(add=True)
                    @pl.when(jj >= drain)
                    def _():
                        sca(jj - drain, (q - drain) % ring).wait()
                    if use_table:
                        @pl.when(jj + lead < NCHUNK)
                        def _():
                            gat(h, jj + lead, (q + lead) % ring).start()
                return 0
            lax.fori_loop(0, NCHUNK // ring, rev_body, 0)
            for k in range(drain):
                jj = NCHUNK - drain + k
                sca(jj, jj % ring).wait()
            plsc.subcore_barrier()

            for k in range(NRCH):
                pltpu.sync_copy(acc_sh.at[pl.ds(base + k * RCH, RCH)], chunk_v)
                pltpu.sync_copy(chunk_v,
                                out_hbm.at[c, h, pl.ds(base + k * RCH, RCH)])
            if h + 1 < nhalf:
                _zero_rows(chunk_v, RCH, d)

    return functools.partial(
        pl.kernel, body, mesh=_MESH,
        out_type=jax.ShapeDtypeStruct((NC, nhalf, NPAD, d), jnp.float32),
        scratch_types=scratch,
        compiler_params=pltpu.CompilerParams(use_tc_tiling_on_sc=False),
    )()


def _dinv_col(degp):
    deg = degp[0, 0] + degp[1, 0] + 1.0
    return lax.rsqrt(deg)[:N, 0:1]


def _tc1_body(x_ref, w1_ref, degp_ref, lo_ref, hi_ref):
    dinv = _dinv_col(degp_ref[...])
    th1 = dinv * jnp.dot(x_ref[...], w1_ref[...],
                         preferred_element_type=jnp.float32)
    lo_ref[...] = th1[:, :D_HID // 2]
    hi_ref[...] = th1[:, D_HID // 2:]


def _tc2_body(acc_ref, lo_ref, hi_ref, degp_ref, b1_ref, gamma_ref,
              beta_ref, w2_ref, th2_ref):
    dinv = _dinv_col(degp_ref[...])
    accsum = jnp.concatenate(
        [acc_ref[0, 0, :N, :] + acc_ref[1, 0, :N, :] + lo_ref[...],
         acc_ref[0, 1, :N, :] + acc_ref[1, 1, :N, :] + hi_ref[...]],
        axis=-1)
    h1 = dinv * accsum
    h1 = h1 + b1_ref[...]
    mean = jnp.mean(h1, axis=0)
    var = jnp.mean((h1 - mean) ** 2, axis=0)
    z = (h1 - mean) * lax.rsqrt(var + 1e-5) * gamma_ref[...] + beta_ref[...]
    z = jnp.maximum(z, 0.0)
    th2_ref[...] = dinv * jnp.dot(z, w2_ref[...],
                                  preferred_element_type=jnp.float32)


def _tc3_body(acc_ref, th2_ref, degp_ref, b2_ref, out_ref):
    dinv = _dinv_col(degp_ref[...])
    h2 = dinv * (acc_ref[0, 0, :N, :] + acc_ref[1, 0, :N, :] + th2_ref[...])
    h2 = h2 + b2_ref[...]
    e = jnp.exp(h2 - jnp.max(h2, axis=-1, keepdims=True))
    out_ref[...] = e / jnp.sum(e, axis=-1, keepdims=True)


def kernel(x, edge_index, W1, b1, gamma, beta, W2, b2):
    er = edge_index.reshape(2, NW, NCHUNK, CH)

    degp = _make_agg(16, 1, use_table=False, ring=8, lead=4)(er)

    th1_lo, th1_hi = pl.pallas_call(
        _tc1_body,
        out_shape=[jax.ShapeDtypeStruct((N, D_HID // 2), jnp.float32),
                   jax.ShapeDtypeStruct((N, D_HID // 2), jnp.float32)],
    )(x, W1, degp)

    acc1 = _make_agg(D_HID // 2, 2, use_table=True, ring=5, lead=4)(
        th1_lo, th1_hi, er)

    th2 = pl.pallas_call(
        _tc2_body,
        out_shape=jax.ShapeDtypeStruct((N, K), jnp.float32),
    )(acc1, th1_lo, th1_hi, degp, b1, gamma, beta, W2)

    acc2 = _make_agg(K, 1, use_table=True, ring=8, lead=6)(th2, er)

    out = pl.pallas_call(
        _tc3_body,
        out_shape=jax.ShapeDtypeStruct((N, K), jnp.float32),
    )(acc2, th2, degp, b2)

    return out
